# baseline (speedup 1.0000x reference)
import jax
import jax.numpy as jnp
from jax import lax
from jax.experimental import pallas as pl
from jax.experimental.pallas import tpu as pltpu
from jax.experimental.pallas import tpu_sc as plsc

B = 16384
L = 20
D = 64
LANES = 16
NC, NS = 2, 16
NW = NC * NS
BPW = B // NW
CB = 32
NCHUNK = BPW // CB
IDX_PER_CHUNK = CB * L
GRANULE = 128
NGATHER = IDX_PER_CHUNK // GRANULE


def _cbow_body(os_hbm, nodes_hbm, table_hbm, y_hbm,
               idx_v, nodes_v, g_v, nrows_v, out_v, gsem, nsem):
    wid = lax.axis_index("s") * NC + lax.axis_index("c")
    base = wid * BPW

    pltpu.sync_copy(os_hbm.at[pl.ds(base * L, BPW * L)], idx_v)
    pltpu.sync_copy(nodes_hbm.at[pl.ds(base, BPW)], nodes_v)

    def chunk_body(c, carry):
        ncopy = pltpu.async_copy(
            table_hbm.at[nodes_v.at[pl.ds(c * CB, CB)]], nrows_v, nsem)
        copies = []
        for g in range(NGATHER):
            copies.append(pltpu.async_copy(
                table_hbm.at[idx_v.at[pl.ds(c * IDX_PER_CHUNK + g * GRANULE,
                                            GRANULE)]],
                g_v.at[pl.ds(g * GRANULE, GRANULE)],
                gsem))
        for cp in copies:
            cp.wait()
        ncopy.wait()

        lane_ids = jnp.arange(LANES, dtype=jnp.int32)

        def lane_sum(v):
            for sh in (8, 4, 2, 1):
                idx = lane_ids ^ sh
                v = v + v.at[idx].get(mode="promise_in_bounds")
            return v

        def grp_body(gi, carry2):
            def lane_body(lane, vec):
                b = gi * LANES + lane
                row = b * L
                t = jnp.zeros((LANES,), jnp.float32)
                for k in range(D // LANES):
                    acc = g_v[row, pl.ds(k * LANES, LANES)]
                    for j in range(1, L):
                        acc = acc + g_v[row + j, pl.ds(k * LANES, LANES)]
                    t = t + acc * nrows_v[b, pl.ds(k * LANES, LANES)]
                s = lane_sum(t) * (1.0 / L)
                return jnp.where(lane_ids == lane, s, vec)

            vec = lax.fori_loop(0, LANES, lane_body,
                                jnp.zeros((LANES,), jnp.float32))
            off = pl.multiple_of(c * CB + gi * LANES, LANES)
            out_v[pl.ds(off, LANES)] = vec
            return carry2

        return lax.fori_loop(0, CB // LANES, grp_body, carry)

    lax.fori_loop(0, NCHUNK, chunk_body, 0)

    def sig_body(i, carry):
        off = pl.multiple_of(i * LANES, LANES)
        v = out_v[pl.ds(off, LANES)]
        out_v[pl.ds(off, LANES)] = 1.0 / (1.0 + jnp.exp(-v))
        return carry

    lax.fori_loop(0, BPW // LANES, sig_body, 0)
    pltpu.sync_copy(out_v, y_hbm.at[pl.ds(base, BPW)])


def kernel(os, nodes, node_embs):
    os_flat = os.reshape(-1)
    mesh = plsc.VectorSubcoreMesh(core_axis_name="c", subcore_axis_name="s")
    run = pl.kernel(
        _cbow_body,
        mesh=mesh,
        out_type=jax.ShapeDtypeStruct((B,), jnp.float32),
        scratch_types=[
            pltpu.VMEM((BPW * L,), jnp.int32),
            pltpu.VMEM((BPW,), jnp.int32),
            pltpu.VMEM((IDX_PER_CHUNK, D), jnp.float32),
            pltpu.VMEM((CB, D), jnp.float32),
            pltpu.VMEM((BPW,), jnp.float32),
            pltpu.SemaphoreType.DMA,
            pltpu.SemaphoreType.DMA,
        ],
        compiler_params=pltpu.CompilerParams(use_tc_tiling_on_sc=False),
    )
    return run(os_flat, nodes, node_embs)
